# baseline (speedup 1.0000x reference)
import functools

import jax
import jax.numpy as jnp
from jax import lax
from jax.experimental import pallas as pl
from jax.experimental.pallas import tpu as pltpu
from jax.experimental.pallas import tpu_sc as plsc

BATCH = 16384
HIST = 50
EMBED_DIM = 32
NUM_CORES = 2
NUM_SUBCORES = 16
NW = NUM_CORES * NUM_SUBCORES
BB = 128
NBB = BATCH // BB
BPW = NBB // NW
ROWS = BPW * BB
NITEM = HIST

_mesh = plsc.VectorSubcoreMesh(core_axis_name="c", subcore_axis_name="s")


NBLK = 7813
TROWS = 250016


@functools.partial(
    pl.kernel,
    mesh=_mesh,
    out_type=(
        jax.ShapeDtypeStruct((BATCH * HIST,), jnp.int32),
        jax.ShapeDtypeStruct((TROWS, 128), jnp.float32),
    ),
    scratch_types=[
        pltpu.VMEM((7, 8, 512), jnp.int32),
        pltpu.VMEM((2, 32, 129), jnp.float32),
        pltpu.VMEM((2, 32, 128), jnp.float32),
        pltpu.SemaphoreType.DMA((7,)),
        pltpu.SemaphoreType.DMA((7,)),
        pltpu.SemaphoreType.DMA((2,)),
        pltpu.SemaphoreType.DMA((2,)),
    ],
    compiler_params=pltpu.CompilerParams(
        use_tc_tiling_on_sc=True, needs_layout_passes=False,
        disable_bounds_checks=True,
    ),
)
def _fmt(xt_hbm, tt_hbm, xl_hbm, trm_hbm, vbuf, sbuf, obuf,
         xrsem, xwsem, trsem, twsem):
    wid = lax.axis_index("s") * NUM_CORES + lax.axis_index("c")
    col0 = wid * 512

    def x_rd(hb):
        rows = 8 if hb < 6 else 2
        return pltpu.make_async_copy(
            xt_hbm.at[pl.ds(hb * 8, rows), pl.ds(col0, 512)],
            vbuf.at[hb, pl.ds(0, rows)],
            xrsem.at[hb],
        )

    def x_wr(h):
        hb, r = divmod(h, 8)
        return pltpu.make_async_copy(
            vbuf.at[hb, r],
            xl_hbm.at[pl.ds(h * BATCH + col0, 512)],
            xwsem.at[hb],
        )

    for hb in range(7):
        x_rd(hb).start()
    for hb in range(7):
        x_rd(hb).wait()
        for r in range(8 if hb < 6 else 2):
            x_wr(hb * 8 + r).start()

    lanes = lax.iota(jnp.int32, 16)
    hi_lanes = lanes + 16
    nblk = 244 + jnp.where(wid < 5, 1, 0)

    def t_rd(rb, s):
        return pltpu.make_async_copy(
            tt_hbm.at[:, pl.ds(rb * 128, 128)],
            sbuf.at[s, :, pl.ds(0, 128)],
            trsem.at[s],
        )

    def t_wr(rb, s):
        return pltpu.make_async_copy(
            obuf.at[s],
            trm_hbm.at[pl.ds(rb * 32, 32)],
            twsem.at[s],
        )

    def transpose_blk(s):
        def b_body(b4, _):
            ba = jnp.full((16,), b4 * 4, jnp.int32)
            for u in range(4):
                v_lo = plsc.load_gather(sbuf.at[s], [lanes, ba + u])
                v_hi = plsc.load_gather(sbuf.at[s], [hi_lanes, ba + u])
                obuf[s, b4, pl.ds(u * 32, 16)] = v_lo
                obuf[s, b4, pl.ds(u * 32 + 16, 16)] = v_hi
            return 0

        lax.fori_loop(0, 32, b_body, 0)

    def rb_of(i):
        return wid + 32 * i

    t_rd(rb_of(0), 0).start()

    def blk_body(i, _):
        s = lax.rem(i, 2)
        ns = lax.rem(i + 1, 2)
        rb = rb_of(i)
        t_rd(rb, s).wait()

        @pl.when(i < nblk - 1)
        def _():
            t_rd(rb_of(i + 1), ns).start()

        @pl.when(i >= 2)
        def _():
            t_wr(rb_of(i - 2), s).wait()

        transpose_blk(s)
        t_wr(rb, s).start()
        return 0

    lax.fori_loop(0, nblk, blk_body, 0)
    t_wr(rb_of(nblk - 2), lax.rem(nblk - 2, 2)).wait()
    t_wr(rb_of(nblk - 1), lax.rem(nblk - 1, 2)).wait()

    for h in range(HIST):
        x_wr(h).wait()


@functools.partial(
    pl.kernel,
    mesh=_mesh,
    out_type=jax.ShapeDtypeStruct((HIST, 4, NBB, 8, BB), jnp.float32),
    scratch_types=[
        pltpu.VMEM((2, ROWS), jnp.int32),
        pltpu.VMEM((2, ROWS, EMBED_DIM), jnp.float32),
        pltpu.VMEM((2, 16, 10, 129), jnp.float32),
        pltpu.SemaphoreType.DMA((2,)),
        pltpu.SemaphoreType.DMA((2,)),
        pltpu.SemaphoreType.DMA((2,)),
    ],
    compiler_params=pltpu.CompilerParams(
        use_tc_tiling_on_sc=False, needs_layout_passes=False
    ),
)
def _emb_lookup(xl_hbm, table_hbm, out_hbm, idx_v, gbuf, obuf, isem, gsem, wsem):
    wid = lax.axis_index("s") * NUM_CORES + lax.axis_index("c")
    col0 = wid * ROWS
    bb0 = wid * BPW

    def idx_copy(t, b):
        return pltpu.async_copy(
            xl_hbm.at[pl.ds(t * BATCH + col0, ROWS)], idx_v.at[b], isem.at[b]
        )

    def gather_copy(b):
        return pltpu.async_copy(table_hbm.at[idx_v.at[b]], gbuf.at[b], gsem.at[b])

    def write_copies(t, b, do_issue):
        for cb in range(4):
            cp = pltpu.make_async_copy(
                obuf.at[b, pl.ds(cb * BPW, BPW), pl.ds(0, 8), pl.ds(0, BB)],
                out_hbm.at[t, cb, pl.ds(bb0, BPW)],
                wsem.at[b],
            )
            if do_issue:
                cp.start()
            else:
                cp.wait()

    lanes = lax.iota(jnp.int32, 16)
    cb4_lo = (lanes // 8) * BPW
    cb4_hi = ((lanes + 16) // 8) * BPW
    ci_vec = lax.rem(lanes, 8)

    def transpose_item(b):
        src = gbuf.at[b]
        dst = obuf.at[b]

        def blk_body(rb, _):
            for u in range(8):
                r = rb * 8 + u
                j = r // BB
                bi = lax.rem(r, BB)
                ja = jnp.full((16,), j, jnp.int32)
                bia = jnp.full((16,), bi, jnp.int32)
                v_lo = src[r, pl.ds(0, 16)]
                v_hi = src[r, pl.ds(16, 16)]
                plsc.store_scatter(dst, [cb4_lo + ja, ci_vec, bia], v_lo)
                plsc.store_scatter(dst, [cb4_hi + ja, ci_vec, bia], v_hi)
            return 0

        lax.fori_loop(0, ROWS // 8, blk_body, 0)

    idx_copy(0, 0).wait()
    gather_copy(0)
    idx_copy(1, 1)

    def body(t, _):
        b = lax.rem(t, 2)
        nb = lax.rem(t + 1, 2)

        pltpu.make_async_copy(
            table_hbm.at[idx_v.at[b]], gbuf.at[b], gsem.at[b]
        ).wait()

        @pl.when(t < NITEM - 2)
        def _():
            idx_copy(t + 2, b)

        @pl.when(t < NITEM - 1)
        def _():
            pltpu.make_async_copy(
                xl_hbm.at[pl.ds((t + 1) * BATCH + col0, ROWS)],
                idx_v.at[nb],
                isem.at[nb],
            ).wait()
            gather_copy(nb)

        @pl.when(t >= 2)
        def _():
            write_copies(t - 2, b, do_issue=False)

        transpose_item(b)
        write_copies(t, b, do_issue=True)
        return 0

    lax.fori_loop(0, NITEM, body, 0)
    write_copies(NITEM - 2, (NITEM - 2) % 2, do_issue=False)
    write_copies(NITEM - 1, (NITEM - 1) % 2, do_issue=False)


def kernel(x, table):
    xt = x.T.astype(jnp.int32)
    tt = table.T
    xl, trm = _fmt(xt, tt)
    out5 = _emb_lookup(xl, trm.reshape(TROWS * 4, EMBED_DIM))
    return out5.transpose(2, 4, 0, 1, 3).reshape(BATCH, HIST, EMBED_DIM)


# speedup vs baseline: 2.0219x; 2.0219x over previous
import functools

import jax
import jax.numpy as jnp
from jax import lax
from jax.experimental import pallas as pl
from jax.experimental.pallas import tpu as pltpu
from jax.experimental.pallas import tpu_sc as plsc

BATCH = 16384
HIST = 50
EMBED_DIM = 32
NUM_CORES = 2
NUM_SUBCORES = 16
NW = NUM_CORES * NUM_SUBCORES
BB = 128
NBB = BATCH // BB
BPW = NBB // NW
ROWS = BPW * BB
NITEM = HIST

_mesh = plsc.VectorSubcoreMesh(core_axis_name="c", subcore_axis_name="s")


NBLK = 7813
TROWS = 250016


@functools.partial(
    pl.kernel,
    mesh=_mesh,
    out_type=(
        jax.ShapeDtypeStruct((BATCH * HIST,), jnp.int32),
        jax.ShapeDtypeStruct((TROWS, 128), jnp.float32),
    ),
    scratch_types=[
        pltpu.VMEM((7, 8, 512), jnp.int32),
        pltpu.VMEM((2, 32, 128), jnp.float32),
        pltpu.VMEM((2, 32, 128), jnp.float32),
        pltpu.SemaphoreType.DMA((7,)),
        pltpu.SemaphoreType.DMA((7,)),
        pltpu.SemaphoreType.DMA((2,)),
        pltpu.SemaphoreType.DMA((2,)),
    ],
    compiler_params=pltpu.CompilerParams(
        use_tc_tiling_on_sc=True, needs_layout_passes=False,
        disable_bounds_checks=True,
    ),
)
def _fmt(xt_hbm, tt_hbm, xl_hbm, trm_hbm, vbuf, sbuf, obuf,
         xrsem, xwsem, trsem, twsem):
    wid = lax.axis_index("s") * NUM_CORES + lax.axis_index("c")
    col0 = wid * 512

    def x_rd(hb):
        rows = 8 if hb < 6 else 2
        return pltpu.make_async_copy(
            xt_hbm.at[pl.ds(hb * 8, rows), pl.ds(col0, 512)],
            vbuf.at[hb, pl.ds(0, rows)],
            xrsem.at[hb],
        )

    def x_wr(h):
        hb, r = divmod(h, 8)
        return pltpu.make_async_copy(
            vbuf.at[hb, r],
            xl_hbm.at[pl.ds(h * BATCH + col0, 512)],
            xwsem.at[hb],
        )

    for hb in range(7):
        x_rd(hb).start()
    for hb in range(7):
        x_rd(hb).wait()
        for r in range(8 if hb < 6 else 2):
            x_wr(hb * 8 + r).start()

    lanes = lax.iota(jnp.int32, 16)
    hi_lanes = lanes + 16
    nblk = 244 + jnp.where(wid < 5, 1, 0)

    def t_rd(rb, s):
        return pltpu.make_async_copy(
            tt_hbm.at[:, pl.ds(rb * 128, 128)],
            sbuf.at[s],
            trsem.at[s],
        )

    def t_wr(rb, s):
        return pltpu.make_async_copy(
            obuf.at[s],
            trm_hbm.at[pl.ds(rb * 32, 32)],
            twsem.at[s],
        )

    diag = [lax.rem(lanes + d, 16) for d in range(16)]
    diag_r4 = [d_ // 4 for d_ in diag]
    diag_c_lo = [(lax.rem(d_, 4)) * 32 + lanes for d_ in diag]
    diag_c_hi = [(lax.rem(d_, 4)) * 32 + hi_lanes for d_ in diag]

    def transpose_blk(s):
        def b_body(g, _):
            b0 = g * 16
            b0a = jnp.full((16,), b0, jnp.int32)
            r0a = jnp.full((16,), b0 // 4, jnp.int32)
            for d in range(16):
                bcol = b0a + diag[d]
                orow = r0a + diag_r4[d]
                v_lo = plsc.load_gather(sbuf.at[s], [lanes, bcol])
                plsc.store_scatter(obuf.at[s], [orow, diag_c_lo[d]], v_lo)
                v_hi = plsc.load_gather(sbuf.at[s], [hi_lanes, bcol])
                plsc.store_scatter(obuf.at[s], [orow, diag_c_hi[d]], v_hi)
            return 0

        lax.fori_loop(0, 8, b_body, 0)

    def rb_of(i):
        return wid + 32 * i

    t_rd(rb_of(0), 0).start()

    def blk_body(i, _):
        s = lax.rem(i, 2)
        ns = lax.rem(i + 1, 2)
        rb = rb_of(i)
        t_rd(rb, s).wait()

        @pl.when(i < nblk - 1)
        def _():
            t_rd(rb_of(i + 1), ns).start()

        @pl.when(i >= 2)
        def _():
            t_wr(rb_of(i - 2), s).wait()

        transpose_blk(s)
        t_wr(rb, s).start()
        return 0

    lax.fori_loop(0, nblk, blk_body, 0)
    t_wr(rb_of(nblk - 2), lax.rem(nblk - 2, 2)).wait()
    t_wr(rb_of(nblk - 1), lax.rem(nblk - 1, 2)).wait()

    for h in range(HIST):
        x_wr(h).wait()


@functools.partial(
    pl.kernel,
    mesh=_mesh,
    out_type=jax.ShapeDtypeStruct((HIST, 4, NBB, 8, BB), jnp.float32),
    scratch_types=[
        pltpu.VMEM((2, ROWS), jnp.int32),
        pltpu.VMEM((2, ROWS, EMBED_DIM), jnp.float32),
        pltpu.VMEM((2, 16, 10, 129), jnp.float32),
        pltpu.SemaphoreType.DMA((2,)),
        pltpu.SemaphoreType.DMA((2,)),
        pltpu.SemaphoreType.DMA((2,)),
    ],
    compiler_params=pltpu.CompilerParams(
        use_tc_tiling_on_sc=False, needs_layout_passes=False
    ),
)
def _emb_lookup(xl_hbm, table_hbm, out_hbm, idx_v, gbuf, obuf, isem, gsem, wsem):
    wid = lax.axis_index("s") * NUM_CORES + lax.axis_index("c")
    col0 = wid * ROWS
    bb0 = wid * BPW

    def idx_copy(t, b):
        return pltpu.async_copy(
            xl_hbm.at[pl.ds(t * BATCH + col0, ROWS)], idx_v.at[b], isem.at[b]
        )

    def gather_copy(b):
        return pltpu.async_copy(table_hbm.at[idx_v.at[b]], gbuf.at[b], gsem.at[b])

    def write_copies(t, b, do_issue):
        for cb in range(4):
            cp = pltpu.make_async_copy(
                obuf.at[b, pl.ds(cb * BPW, BPW), pl.ds(0, 8), pl.ds(0, BB)],
                out_hbm.at[t, cb, pl.ds(bb0, BPW)],
                wsem.at[b],
            )
            if do_issue:
                cp.start()
            else:
                cp.wait()

    lanes = lax.iota(jnp.int32, 16)
    cb4_lo = (lanes // 8) * BPW
    cb4_hi = ((lanes + 16) // 8) * BPW
    ci_vec = lax.rem(lanes, 8)

    def transpose_item(b):
        src = gbuf.at[b]
        dst = obuf.at[b]

        def blk_body(rb, _):
            for u in range(8):
                r = rb * 8 + u
                j = r // BB
                bi = lax.rem(r, BB)
                ja = jnp.full((16,), j, jnp.int32)
                bia = jnp.full((16,), bi, jnp.int32)
                v_lo = src[r, pl.ds(0, 16)]
                v_hi = src[r, pl.ds(16, 16)]
                plsc.store_scatter(dst, [cb4_lo + ja, ci_vec, bia], v_lo)
                plsc.store_scatter(dst, [cb4_hi + ja, ci_vec, bia], v_hi)
            return 0

        lax.fori_loop(0, ROWS // 8, blk_body, 0)

    idx_copy(0, 0).wait()
    gather_copy(0)
    idx_copy(1, 1)

    def body(t, _):
        b = lax.rem(t, 2)
        nb = lax.rem(t + 1, 2)

        pltpu.make_async_copy(
            table_hbm.at[idx_v.at[b]], gbuf.at[b], gsem.at[b]
        ).wait()

        @pl.when(t < NITEM - 2)
        def _():
            idx_copy(t + 2, b)

        @pl.when(t < NITEM - 1)
        def _():
            pltpu.make_async_copy(
                xl_hbm.at[pl.ds((t + 1) * BATCH + col0, ROWS)],
                idx_v.at[nb],
                isem.at[nb],
            ).wait()
            gather_copy(nb)

        @pl.when(t >= 2)
        def _():
            write_copies(t - 2, b, do_issue=False)

        transpose_item(b)
        write_copies(t, b, do_issue=True)
        return 0

    lax.fori_loop(0, NITEM, body, 0)
    write_copies(NITEM - 2, (NITEM - 2) % 2, do_issue=False)
    write_copies(NITEM - 1, (NITEM - 1) % 2, do_issue=False)


def kernel(x, table):
    xt = x.T.astype(jnp.int32)
    tt = table.T
    xl, trm = _fmt(xt, tt)
    out5 = _emb_lookup(xl, trm.reshape(TROWS * 4, EMBED_DIM))
    return out5.transpose(2, 4, 0, 1, 3).reshape(BATCH, HIST, EMBED_DIM)


# speedup vs baseline: 2.0432x; 1.0105x over previous
import functools

import jax
import jax.numpy as jnp
from jax import lax
from jax.experimental import pallas as pl
from jax.experimental.pallas import tpu as pltpu
from jax.experimental.pallas import tpu_sc as plsc

BATCH = 16384
HIST = 50
EMBED_DIM = 32
NUM_CORES = 2
NUM_SUBCORES = 16
NW = NUM_CORES * NUM_SUBCORES
BB = 128
NBB = BATCH // BB
BPW = NBB // NW
ROWS = BPW * BB
NITEM = HIST

_mesh = plsc.VectorSubcoreMesh(core_axis_name="c", subcore_axis_name="s")


NBLK = 7813
TROWS = 250016


@functools.partial(
    pl.kernel,
    mesh=_mesh,
    out_type=(
        jax.ShapeDtypeStruct((BATCH * HIST,), jnp.int32),
        jax.ShapeDtypeStruct((TROWS, 128), jnp.float32),
    ),
    scratch_types=[
        pltpu.VMEM((7, 8, 512), jnp.int32),
        pltpu.VMEM((4, 32, 128), jnp.float32),
        pltpu.VMEM((4, 32, 128), jnp.float32),
        pltpu.SemaphoreType.DMA((7,)),
        pltpu.SemaphoreType.DMA((7,)),
        pltpu.SemaphoreType.DMA((4,)),
        pltpu.SemaphoreType.DMA((4,)),
    ],
    compiler_params=pltpu.CompilerParams(
        use_tc_tiling_on_sc=True, needs_layout_passes=False,
        disable_bounds_checks=True,
    ),
)
def _fmt(xt_hbm, tt_hbm, xl_hbm, trm_hbm, vbuf, sbuf, obuf,
         xrsem, xwsem, trsem, twsem):
    wid = lax.axis_index("s") * NUM_CORES + lax.axis_index("c")
    col0 = wid * 512

    def x_rd(hb):
        rows = 8 if hb < 6 else 2
        return pltpu.make_async_copy(
            xt_hbm.at[pl.ds(hb * 8, rows), pl.ds(col0, 512)],
            vbuf.at[hb, pl.ds(0, rows)],
            xrsem.at[hb],
        )

    def x_wr(h):
        hb, r = divmod(h, 8)
        return pltpu.make_async_copy(
            vbuf.at[hb, r],
            xl_hbm.at[pl.ds(h * BATCH + col0, 512)],
            xwsem.at[hb],
        )

    for hb in range(7):
        x_rd(hb).start()
    for hb in range(7):
        x_rd(hb).wait()
        for r in range(8 if hb < 6 else 2):
            x_wr(hb * 8 + r).start()

    lanes = lax.iota(jnp.int32, 16)
    hi_lanes = lanes + 16
    nblk = 244 + jnp.where(wid < 5, 1, 0)

    def t_rd(rb, s):
        return pltpu.make_async_copy(
            tt_hbm.at[:, pl.ds(rb * 128, 128)],
            sbuf.at[s],
            trsem.at[s],
        )

    def t_wr(rb, s):
        return pltpu.make_async_copy(
            obuf.at[s],
            trm_hbm.at[pl.ds(rb * 32, 32)],
            twsem.at[s],
        )

    diag = [lax.rem(lanes + d, 16) for d in range(16)]
    diag_r4 = [d_ // 4 for d_ in diag]
    diag_c_lo = [(lax.rem(d_, 4)) * 32 + lanes for d_ in diag]
    diag_c_hi = [(lax.rem(d_, 4)) * 32 + hi_lanes for d_ in diag]

    def transpose_blk(s):
        def b_body(g, _):
            b0 = g * 16
            b0a = jnp.full((16,), b0, jnp.int32)
            r0a = jnp.full((16,), b0 // 4, jnp.int32)
            for d in range(16):
                bcol = b0a + diag[d]
                orow = r0a + diag_r4[d]
                v_lo = plsc.load_gather(sbuf.at[s], [lanes, bcol])
                plsc.store_scatter(obuf.at[s], [orow, diag_c_lo[d]], v_lo)
                v_hi = plsc.load_gather(sbuf.at[s], [hi_lanes, bcol])
                plsc.store_scatter(obuf.at[s], [orow, diag_c_hi[d]], v_hi)
            return 0

        lax.fori_loop(0, 8, b_body, 0)

    def rb_of(i):
        return wid + 32 * i

    for p in range(3):
        t_rd(rb_of(p), p).start()

    def blk_body(i, _):
        s = lax.rem(i, 4)
        rb = rb_of(i)
        t_rd(rb, s).wait()

        @pl.when(i < nblk - 3)
        def _():
            t_rd(rb_of(i + 3), lax.rem(i + 3, 4)).start()

        @pl.when(i >= 4)
        def _():
            t_wr(rb_of(i - 4), s).wait()

        transpose_blk(s)
        t_wr(rb, s).start()
        return 0

    lax.fori_loop(0, nblk, blk_body, 0)
    for p in range(4):
        t_wr(rb_of(nblk - 4 + p), lax.rem(nblk - 4 + p, 4)).wait()

    for h in range(HIST):
        x_wr(h).wait()


@functools.partial(
    pl.kernel,
    mesh=_mesh,
    out_type=jax.ShapeDtypeStruct((HIST, 4, NBB, 8, BB), jnp.float32),
    scratch_types=[
        pltpu.VMEM((2, ROWS), jnp.int32),
        pltpu.VMEM((2, ROWS, EMBED_DIM), jnp.float32),
        pltpu.VMEM((2, 16, 10, 129), jnp.float32),
        pltpu.SemaphoreType.DMA((2,)),
        pltpu.SemaphoreType.DMA((2,)),
        pltpu.SemaphoreType.DMA((2,)),
    ],
    compiler_params=pltpu.CompilerParams(
        use_tc_tiling_on_sc=False, needs_layout_passes=False
    ),
)
def _emb_lookup(xl_hbm, table_hbm, out_hbm, idx_v, gbuf, obuf, isem, gsem, wsem):
    wid = lax.axis_index("s") * NUM_CORES + lax.axis_index("c")
    col0 = wid * ROWS
    bb0 = wid * BPW

    def idx_copy(t, b):
        return pltpu.async_copy(
            xl_hbm.at[pl.ds(t * BATCH + col0, ROWS)], idx_v.at[b], isem.at[b]
        )

    def gather_copy(b):
        return pltpu.async_copy(table_hbm.at[idx_v.at[b]], gbuf.at[b], gsem.at[b])

    def write_copies(t, b, do_issue):
        for cb in range(4):
            cp = pltpu.make_async_copy(
                obuf.at[b, pl.ds(cb * BPW, BPW), pl.ds(0, 8), pl.ds(0, BB)],
                out_hbm.at[t, cb, pl.ds(bb0, BPW)],
                wsem.at[b],
            )
            if do_issue:
                cp.start()
            else:
                cp.wait()

    lanes = lax.iota(jnp.int32, 16)
    cb4_lo = (lanes // 8) * BPW
    cb4_hi = ((lanes + 16) // 8) * BPW
    ci_vec = lax.rem(lanes, 8)

    def transpose_item(b):
        src = gbuf.at[b]
        dst = obuf.at[b]

        def blk_body(rb, _):
            for u in range(8):
                r = rb * 8 + u
                j = r // BB
                bi = lax.rem(r, BB)
                ja = jnp.full((16,), j, jnp.int32)
                bia = jnp.full((16,), bi, jnp.int32)
                v_lo = src[r, pl.ds(0, 16)]
                v_hi = src[r, pl.ds(16, 16)]
                plsc.store_scatter(dst, [cb4_lo + ja, ci_vec, bia], v_lo)
                plsc.store_scatter(dst, [cb4_hi + ja, ci_vec, bia], v_hi)
            return 0

        lax.fori_loop(0, ROWS // 8, blk_body, 0)

    idx_copy(0, 0).wait()
    gather_copy(0)
    idx_copy(1, 1)

    def body(t, _):
        b = lax.rem(t, 2)
        nb = lax.rem(t + 1, 2)

        pltpu.make_async_copy(
            table_hbm.at[idx_v.at[b]], gbuf.at[b], gsem.at[b]
        ).wait()

        @pl.when(t < NITEM - 2)
        def _():
            idx_copy(t + 2, b)

        @pl.when(t < NITEM - 1)
        def _():
            pltpu.make_async_copy(
                xl_hbm.at[pl.ds((t + 1) * BATCH + col0, ROWS)],
                idx_v.at[nb],
                isem.at[nb],
            ).wait()
            gather_copy(nb)

        @pl.when(t >= 2)
        def _():
            write_copies(t - 2, b, do_issue=False)

        transpose_item(b)
        write_copies(t, b, do_issue=True)
        return 0

    lax.fori_loop(0, NITEM, body, 0)
    write_copies(NITEM - 2, (NITEM - 2) % 2, do_issue=False)
    write_copies(NITEM - 1, (NITEM - 1) % 2, do_issue=False)


def kernel(x, table):
    xt = x.T.astype(jnp.int32)
    tt = table.T
    xl, trm = _fmt(xt, tt)
    out5 = _emb_lookup(xl, trm.reshape(TROWS * 4, EMBED_DIM))
    return out5.transpose(2, 4, 0, 1, 3).reshape(BATCH, HIST, EMBED_DIM)
